# baseline (speedup 1.0000x reference)
import functools

import jax
import jax.numpy as jnp
from jax import lax
from jax.experimental import pallas as pl
from jax.experimental.pallas import tpu as pltpu
from jax.experimental.pallas import tpu_sc as plsc

N_NODES = 10000
N_EDGES = 320000
D = 128

NC = 2
NS = 16
NW = NC * NS

CHUNK = 128
CPT = 80
E_PAD = NW * CPT * CHUNK
N_PAD = 10240
ROWS_PER_TILE = N_PAD // NS
HALF = CPT // 2


def _sc_body(src_hbm, dst_hbm, x_hbm, part_hbm, acc_sh, sbuf, dbuf,
             rows0, rows1, sem0, sem1):
    c = lax.axis_index("c")
    s = lax.axis_index("s")
    wid = s * NC + c

    def _zero(t, carry):
        i = t // 8
        j = t % 8
        rows0[i, pl.ds(j * 16, 16)] = jnp.zeros((16,), jnp.float32)
        return carry

    lax.fori_loop(0, CHUNK * 8, _zero, None)
    for j in range(ROWS_PER_TILE // CHUNK):
        pltpu.sync_copy(rows0, acc_sh.at[pl.ds(s * ROWS_PER_TILE + j * CHUNK,
                                               CHUNK)])
    plsc.subcore_barrier()

    start = wid * CPT
    for h in range(2):
        pltpu.sync_copy(src_hbm.at[pl.ds(start + h * HALF, HALF)], sbuf)
        pltpu.sync_copy(dst_hbm.at[pl.ds(start + h * HALF, HALF)], dbuf)

        pltpu.async_copy(x_hbm.at[sbuf.at[0]], rows0, sem0)
        pltpu.async_copy(x_hbm.at[sbuf.at[1]], rows1, sem1)

        def _pair(g, carry):
            k0 = g * 2
            pltpu.make_async_copy(x_hbm.at[pl.ds(0, CHUNK)], rows0,
                                  sem0).wait()
            pltpu.sync_copy(rows0, acc_sh.at[dbuf.at[k0]], add=True)
            pltpu.async_copy(x_hbm.at[sbuf.at[jnp.minimum(k0 + 2, HALF - 1)]],
                             rows0, sem0)
            pltpu.make_async_copy(x_hbm.at[pl.ds(0, CHUNK)], rows1,
                                  sem1).wait()
            pltpu.sync_copy(rows1, acc_sh.at[dbuf.at[k0 + 1]], add=True)
            pltpu.async_copy(x_hbm.at[sbuf.at[jnp.minimum(k0 + 3, HALF - 1)]],
                             rows1, sem1)
            return carry

        lax.fori_loop(0, HALF // 2, _pair, None)
        pltpu.make_async_copy(x_hbm.at[pl.ds(0, CHUNK)], rows0, sem0).wait()
        pltpu.make_async_copy(x_hbm.at[pl.ds(0, CHUNK)], rows1, sem1).wait()
    plsc.subcore_barrier()

    pltpu.sync_copy(acc_sh.at[pl.ds(s * ROWS_PER_TILE, ROWS_PER_TILE)],
                    part_hbm.at[c, pl.ds(s * ROWS_PER_TILE, ROWS_PER_TILE)])


@jax.jit
def _sc_scatter(src2d, dst2d, x):
    mesh = plsc.VectorSubcoreMesh(core_axis_name="c", subcore_axis_name="s")
    return pl.kernel(
        _sc_body,
        mesh=mesh,
        out_type=jax.ShapeDtypeStruct((NC, N_PAD, D), jnp.float32),
        scratch_types=[
            pltpu.VMEM_SHARED((N_PAD, D), jnp.float32),
            pltpu.VMEM((HALF, CHUNK), jnp.int32),
            pltpu.VMEM((HALF, CHUNK), jnp.int32),
            pltpu.VMEM((CHUNK, D), jnp.float32),
            pltpu.VMEM((CHUNK, D), jnp.float32),
            pltpu.SemaphoreType.DMA,
            pltpu.SemaphoreType.DMA,
        ],
    )(src2d, dst2d, x)


def _mm_body(p_ref, w_ref, b_ref, o_ref):
    h = p_ref[0] + p_ref[1]
    o_ref[...] = lax.dot_general(
        h, w_ref[...], (((1,), (1,)), ((), ())),
        preferred_element_type=jnp.float32) + b_ref[...]


def _tc_linear(parts, W, b2d):
    bn = 1000
    grid = N_NODES // bn
    return pl.pallas_call(
        _mm_body,
        grid=(grid,),
        in_specs=[
            pl.BlockSpec((NC, bn, D), lambda i: (0, i, 0)),
            pl.BlockSpec((D, D), lambda i: (0, 0)),
            pl.BlockSpec((1, D), lambda i: (0, 0)),
        ],
        out_specs=pl.BlockSpec((bn, D), lambda i: (i, 0)),
        out_shape=jax.ShapeDtypeStruct((N_NODES, D), jnp.float32),
    )(parts, W, b2d)


def kernel(x, edge_index, W, b):
    src = edge_index[0]
    dst = edge_index[1]
    npad = E_PAD - N_EDGES
    src2d = jnp.concatenate(
        [src, jnp.zeros((npad,), jnp.int32)]).reshape(E_PAD // CHUNK, CHUNK)
    dst2d = jnp.concatenate(
        [dst, jnp.full((npad,), N_NODES, jnp.int32)]).reshape(
            E_PAD // CHUNK, CHUNK)
    parts = _sc_scatter(src2d, dst2d, x)
    return _tc_linear(parts, W, b.reshape(1, D))
